# Initial kernel scaffold; baseline (speedup 1.0000x reference)
import jax
import jax.numpy as jnp
from jax.experimental import pallas as pl


def kernel(x, edge_index, Wq, Wk, Wv, Wo, bo, gamma, beta):
    raise NotImplementedError("write your pallas kernel here")


# speedup vs baseline: 2.5055x; 2.5055x over previous
import functools

import jax
import jax.numpy as jnp
import numpy as np
from jax import lax
from jax.experimental import pallas as pl
from jax.experimental.pallas import tpu as pltpu
from jax.experimental.pallas import tpu_sc as plsc

NC = 2
NS = 16
NW = NC * NS
EB = 64

_INTERPRET = False


def _round_up(a, b):
    return (a + b - 1) // b * b


def _block_rows(n, cap=1024):
    for c in (1024, 1000, 512, 500, 256, 250, 128, 125, 64, 50, 25, 16, 8):
        if c <= cap and n % c == 0:
            return c
    return n


def _qkv_body(x_ref, wq_ref, wkv_ref, q_ref, kv_ref):
    xb = x_ref[...]
    q_ref[...] = jnp.dot(xb, wq_ref[...], preferred_element_type=jnp.float32)
    kv_ref[...] = jnp.dot(xb, wkv_ref[...], preferred_element_type=jnp.float32)


def _qkv_project(x_pad, wq_t, wkv_t):
    np_, d = x_pad.shape
    bn = _block_rows(np_)
    return pl.pallas_call(
        _qkv_body,
        grid=(np_ // bn,),
        in_specs=[
            pl.BlockSpec((bn, d), lambda i: (i, 0)),
            pl.BlockSpec((d, d), lambda i: (0, 0)),
            pl.BlockSpec((d, 2 * d), lambda i: (0, 0)),
        ],
        out_specs=[
            pl.BlockSpec((bn, d), lambda i: (i, 0)),
            pl.BlockSpec((bn, 2 * d), lambda i: (i, 0)),
        ],
        out_shape=[
            jax.ShapeDtypeStruct((np_, d), jnp.float32),
            jax.ShapeDtypeStruct((np_, 2 * d), jnp.float32),
        ],
        interpret=_INTERPRET,
    )(x_pad, wq_t, wkv_t)


_GATHER_DN = lax.GatherDimensionNumbers(
    offset_dims=(), collapsed_slice_dims=(0,), start_index_map=(0,))


def _butterfly_sum(v):
    idx = lax.iota(jnp.int32, 16)
    for k in (1, 2, 4, 8):
        perm = jnp.bitwise_xor(idx, k)
        shuf = lax.gather(v, perm[:, None], _GATHER_DN, (1,),
                          mode=lax.GatherScatterMode.PROMISE_IN_BOUNDS)
        v = v + shuf
    return v

def _make_edge_body(d, nbatch, rows_per_tile):
    def _edge_body(q_hbm, kv_hbm, src_hbm, dst_hbm, out_hbm,
                   src_v, dst_v, qbuf, kvbuf, acc, gsem, isem):
        cid = lax.axis_index("c")
        sid = lax.axis_index("s")
        wid = sid * NC + cid

        def zstep(i, c):
            qbuf[i // (d // 16), pl.ds(16 * (i % (d // 16)), 16)] = (
                jnp.zeros((16,), jnp.float32))
            return c

        lax.fori_loop(0, EB * (d // 16), zstep, 0)
        tbase = sid * rows_per_tile
        for c in range(rows_per_tile // EB):
            pltpu.sync_copy(qbuf, acc.at[pl.ds(tbase + c * EB, EB)])

        plsc.subcore_barrier()

        def batch_step(j, carry):
            cp_s = pltpu.async_copy(src_hbm.at[wid, pl.ds(j, 1)], src_v, isem)
            cp_d = pltpu.async_copy(dst_hbm.at[wid, pl.ds(j, 1)], dst_v, isem)
            cp_s.wait()
            cp_d.wait()
            cp_q = pltpu.async_copy(q_hbm.at[dst_v.at[0]], qbuf, gsem)
            cp_kv = pltpu.async_copy(kv_hbm.at[src_v.at[0]], kvbuf, gsem)
            cp_q.wait()
            cp_kv.wait()

            def edge_step(e, carry2):
                dot = jnp.zeros((16,), jnp.float32)
                for i in range(d // 16):
                    qv = qbuf[e, pl.ds(16 * i, 16)]
                    kv = kvbuf[e, pl.ds(16 * i, 16)]
                    dot = dot + qv * kv
                sv = _butterfly_sum(dot)
                alpha = 1.0 / (1.0 + jnp.exp(-sv))
                for i in range(d // 16):
                    vv = kvbuf[e, pl.ds(d + 16 * i, 16)]
                    qbuf[e, pl.ds(16 * i, 16)] = alpha * vv
                return carry2

            lax.fori_loop(0, EB, edge_step, 0, unroll=2)

            pltpu.sync_copy(qbuf, acc.at[dst_v.at[0]], add=True)
            return carry

        lax.fori_loop(0, nbatch, batch_step, 0)

        plsc.subcore_barrier()

        pltpu.sync_copy(acc.at[pl.ds(tbase, rows_per_tile)],
                        out_hbm.at[cid, pl.ds(tbase, rows_per_tile)])

    return _edge_body


@functools.cache
def _make_edge_kernel(np_, d, nbatch):
    rows_per_tile = np_ // NS
    return pl.kernel(
        _make_edge_body(d, nbatch, rows_per_tile),
        out_type=jax.ShapeDtypeStruct((NC, np_, d), jnp.float32),
        mesh=plsc.VectorSubcoreMesh(core_axis_name="c", subcore_axis_name="s",
                                    num_cores=NC, num_subcores=NS),
        scratch_types=[
            pltpu.VMEM((1, EB), jnp.int32),
            pltpu.VMEM((1, EB), jnp.int32),
            pltpu.VMEM((EB, d), jnp.float32),
            pltpu.VMEM((EB, 2 * d), jnp.float32),
            pltpu.VMEM_SHARED((np_, d), jnp.float32),
            pltpu.SemaphoreType.DMA,
            pltpu.SemaphoreType.DMA,
        ],
        interpret=_INTERPRET,
    )


def _fin_body(p0_ref, p1_ref, x_ref, wo_ref, bo_ref, g_ref, b_ref, o_ref):
    s = p0_ref[...] + p1_ref[...]
    out = jnp.dot(s, wo_ref[...], preferred_element_type=jnp.float32)
    h = x_ref[...] + out + bo_ref[...]
    mu = jnp.mean(h, axis=-1, keepdims=True)
    c = h - mu
    var = jnp.mean(c * c, axis=-1, keepdims=True)
    o_ref[...] = c * lax.rsqrt(var + 1e-5) * g_ref[...] + b_ref[...]


def _finalize(p0, p1, x, wo_t, bo, gamma, beta):
    n, d = x.shape
    bn = _block_rows(n)
    return pl.pallas_call(
        _fin_body,
        grid=(n // bn,),
        in_specs=[
            pl.BlockSpec((bn, d), lambda i: (i, 0)),
            pl.BlockSpec((bn, d), lambda i: (i, 0)),
            pl.BlockSpec((bn, d), lambda i: (i, 0)),
            pl.BlockSpec((d, d), lambda i: (0, 0)),
            pl.BlockSpec((1, d), lambda i: (0, 0)),
            pl.BlockSpec((1, d), lambda i: (0, 0)),
            pl.BlockSpec((1, d), lambda i: (0, 0)),
        ],
        out_specs=pl.BlockSpec((bn, d), lambda i: (i, 0)),
        out_shape=jax.ShapeDtypeStruct((n, d), jnp.float32),
        interpret=_INTERPRET,
    )(p0, p1, x, wo_t, bo, gamma, beta)


@jax.jit
def kernel(x, edge_index, Wq, Wk, Wv, Wo, bo, gamma, beta):
    n, d = x.shape
    e = edge_index.shape[1]
    np_ = _round_up(n + 1, NS * EB)
    epw = _round_up(-(-e // NW), EB)
    nbatch = epw // EB

    wq_t = Wq.T * np.float32(1.0 / np.sqrt(d))
    wkv_t = jnp.concatenate([Wk.T, Wv.T], axis=1)
    x_pad = jnp.pad(x, ((0, np_ - n), (0, 0)))

    epad = NW * epw - e
    src_p = jnp.concatenate([edge_index[0], jnp.zeros((epad,), jnp.int32)])
    dst_p = jnp.concatenate([edge_index[1], jnp.full((epad,), n, jnp.int32)])
    src3 = src_p.reshape(NW, nbatch, EB)
    dst3 = dst_p.reshape(NW, nbatch, EB)

    q, kv = _qkv_project(x_pad, wq_t, wkv_t)
    partials = _make_edge_kernel(np_, d, nbatch)(q, kv, src3, dst3)
    return _finalize(partials[0, :n], partials[1, :n], x,
                     Wo.T, bo.reshape(1, d), gamma.reshape(1, d),
                     beta.reshape(1, d))
